# baseline (speedup 1.0000x reference)
import functools

import jax
import jax.numpy as jnp
from jax import lax
from jax.experimental import pallas as pl
from jax.experimental.pallas import tpu as pltpu
from jax.experimental.pallas import tpu_sc as plsc

B, L, E, Mt, Mc, V = 4, 4, 1024, 1024, 512, 50000
F = 3 * E
GAMMA = 0.8
VP = 50176
HALF = VP // 2
NV = HALF // 16
CH = HALF // 2
NCH = CH // 16
BM = 512
BIGI = 2 ** 30


def _k1_body(h_ref, wt_ref, wg_ref, bg_ref, vm_ref, mm_ref,
             hp_ref, gate_ref, vmp_ref, mmp_ref):
    h = h_ref[...]
    hp_ref[...] = jnp.dot(h, wt_ref[...], preferred_element_type=jnp.float32)
    z = jnp.dot(h, wg_ref[...], preferred_element_type=jnp.float32)
    gate_ref[...] = jax.nn.sigmoid(z + bg_ref[0, 0])
    vmp_ref[0, 0, :V] = vm_ref[0, 0]
    vmp_ref[0, 0, V:] = jnp.zeros((VP - V,), jnp.int32)
    mmp_ref[0, 0, :V] = mm_ref[0, 0]
    mmp_ref[0, 0, V:] = jnp.zeros((VP - V,), jnp.int32)


def _k1(h16, W_triple, Wg_t, bg, vm3, mm3):
    return pl.pallas_call(
        _k1_body,
        grid=(B,),
        out_shape=[jax.ShapeDtypeStruct((16, F), jnp.float32),
                   jax.ShapeDtypeStruct((16, 1), jnp.float32),
                   jax.ShapeDtypeStruct((B, 1, VP), jnp.int32),
                   jax.ShapeDtypeStruct((B, 1, VP), jnp.int32)],
        in_specs=[pl.BlockSpec((16, E), lambda b: (0, 0)),
                  pl.BlockSpec((E, F), lambda b: (0, 0)),
                  pl.BlockSpec((E, 1), lambda b: (0, 0)),
                  pl.BlockSpec(memory_space=pltpu.SMEM),
                  pl.BlockSpec((1, 1, V), lambda b: (b, 0, 0)),
                  pl.BlockSpec((1, 1, V), lambda b: (b, 0, 0))],
        out_specs=[pl.BlockSpec((16, F), lambda b: (0, 0)),
                   pl.BlockSpec((16, 1), lambda b: (0, 0)),
                   pl.BlockSpec((1, 1, VP), lambda b: (b, 0, 0)),
                   pl.BlockSpec((1, 1, VP), lambda b: (b, 0, 0))],
    )(h16, W_triple, Wg_t, bg, vm3, mm3)


def _k2_body(hp_ref, tr_ref, lab_ref, out_ref):
    hp = hp_ref[0]
    tr = tr_ref[0]
    logits = lax.dot_general(hp, tr, (((1,), (1,)), ((), ())),
                             preferred_element_type=jnp.float32)
    prob = jax.nn.sigmoid(logits)
    lab = lab_ref[0]
    out_ref[0] = jnp.where(lab == -1, 0.0, prob)


def _k2(hp, triple_repr, labels3):
    return pl.pallas_call(
        _k2_body,
        grid=(B, Mt // BM),
        out_shape=jax.ShapeDtypeStruct((B, L, Mt), jnp.float32),
        in_specs=[pl.BlockSpec((1, L, F), lambda b, j: (b, 0, 0)),
                  pl.BlockSpec((1, BM, F), lambda b, j: (b, j, 0)),
                  pl.BlockSpec((1, 1, BM), lambda b, j: (b, 0, j))],
        out_specs=pl.BlockSpec((1, L, BM), lambda b, j: (b, 0, j)),
    )(hp, triple_repr, labels3)


def _k34_body(tp_hbm, head_hbm, tail_hbm, tlab_hbm, clab_hbm, dist_hbm,
              vm_hbm, mm_hbm,
              cpv_hbm,
              head_v, tail_v, tlab_v, tp_v, clab_v, dist_v, upd_v, ns_v,
              priv_v, acc_v,
              table_v, vm_v, mm_v, cv_v, shared_cp, sem):
    c = lax.axis_index("c")
    s = lax.axis_index("s")
    pair2 = c * 8 + s // 2
    half = s % 2
    b2 = pair2 // 4
    l2 = pair2 % 4
    base = pl.multiple_of(half * HALF, 32)

    @pl.when(s < 8)
    def _():
        pair = c * 8 + s
        b = pair // 4
        l = pair % 4
        d1 = pltpu.async_copy(head_hbm.at[b], head_v, sem)
        d2 = pltpu.async_copy(tail_hbm.at[b], tail_v, sem)
        d3 = pltpu.async_copy(tlab_hbm.at[b], tlab_v, sem)
        d4 = pltpu.async_copy(tp_hbm.at[b, l], tp_v, sem)
        d5 = pltpu.async_copy(clab_hbm.at[b], clab_v, sem)
        d6 = pltpu.async_copy(dist_hbm.at[b], dist_v, sem)
        d1.wait()
        d2.wait()
        d3.wait()
        d4.wait()
        d5.wait()
        d6.wait()
        one = jnp.ones((16,), jnp.float32)
        zero = jnp.zeros((16,), jnp.float32)
        lane = lax.iota(jnp.int32, 16)
        for i in range(Mc // 16):
            sl = pl.ds(i * 16, 16)
            f0 = jnp.where(dist_v[sl] == 0, one, zero)
            acc_v[sl] = f0
            ns_v[sl] = jnp.where(clab_v[sl] == -1, zero, f0)
        for _hop in range(2):
            for i in range(Mt // 16):
                sl = pl.ds(i * 16, 16)
                hs = plsc.load_gather(ns_v, [head_v[sl]])
                upd = hs * GAMMA + tp_v[sl]
                upd_v[sl] = jnp.where(tlab_v[sl] == -1, zero, upd)

            def zbody(i, carry):
                priv_v[pl.ds(i * 16, 16)] = zero
                return carry

            lax.fori_loop(0, 16 * Mc // 16, zbody, 0)

            def scbody(i, carry):
                sl = pl.ds(i * 16, 16)
                addr = lane * Mc + tail_v[sl]
                cur = plsc.load_gather(priv_v, [addr])
                plsc.store_scatter(priv_v, [addr],
                                   jnp.maximum(cur, upd_v[sl]))
                return carry

            lax.fori_loop(0, Mt // 16, scbody, 0)

            def cbody(i, carry):
                sl16 = pl.ds(i * 16, 16)
                m = priv_v[sl16]
                for j in range(1, 16):
                    m = jnp.maximum(m, priv_v[pl.ds(j * Mc + i * 16, 16)])
                o = jnp.where(clab_v[sl16] == -1, zero, m)
                ns_v[sl16] = o
                acc_v[sl16] = acc_v[sl16] + o
                return carry

            lax.fori_loop(0, Mc // 16, cbody, 0)
        mxv = acc_v[pl.ds(0, 16)]
        for i in range(1, Mc // 16):
            mxv = jnp.maximum(mxv, acc_v[pl.ds(i * 16, 16)])
        m = jnp.max(mxv)
        sv = jnp.zeros((16,), jnp.float32)
        for i in range(Mc // 16):
            sl = pl.ds(i * 16, 16)
            e = jnp.exp(acc_v[sl] - m)
            acc_v[sl] = e
            sv = sv + e
        inv_v = one / (zero + jnp.sum(sv))
        for i in range(Mc // 16):
            sl = pl.ds(i * 16, 16)
            acc_v[sl] = acc_v[sl] * inv_v
        pltpu.sync_copy(acc_v, shared_cp.at[s])

    plsc.subcore_barrier()

    pltpu.sync_copy(shared_cp.at[s // 2], table_v)
    zero = jnp.zeros((16,), jnp.float32)

    def inner(k, carry):
        sl = pl.ds(k * 16, 16)
        cpx = plsc.load_gather(table_v, [vm_v[sl]])
        cv_v[sl] = jnp.where(mm_v[sl] == 0, zero, cpx)
        return carry

    d_vm = pltpu.async_copy(vm_hbm.at[b2, pl.ds(base, CH)], vm_v, sem)
    d_mm = pltpu.async_copy(mm_hbm.at[b2, pl.ds(base, CH)], mm_v, sem)
    d_vm.wait()
    d_mm.wait()
    lax.fori_loop(0, NCH, inner, 0)
    pltpu.sync_copy(cv_v, cpv_hbm.at[b2, l2, pl.ds(base, CH)])
    base1 = pl.multiple_of(base + CH, 32)
    d_vm1 = pltpu.async_copy(vm_hbm.at[b2, pl.ds(base1, CH)], vm_v, sem)
    d_mm1 = pltpu.async_copy(mm_hbm.at[b2, pl.ds(base1, CH)], mm_v, sem)
    d_vm1.wait()
    d_mm1.wait()
    lax.fori_loop(0, NCH, inner, 0)
    pltpu.sync_copy(cv_v, cpv_hbm.at[b2, l2, pl.ds(base1, CH)])


def _k34(tp, head_idx, tail_idx, triple_labels, concept_labels, distances,
         vm_p, mm_p):
    mesh = plsc.VectorSubcoreMesh(core_axis_name="c", subcore_axis_name="s")
    f = functools.partial(
        pl.kernel, _k34_body, mesh=mesh,
        compiler_params=pltpu.CompilerParams(needs_layout_passes=False),
        out_type=jax.ShapeDtypeStruct((B, L, VP), jnp.float32),
        scratch_types=[
            pltpu.VMEM((Mt,), jnp.int32),
            pltpu.VMEM((Mt,), jnp.int32),
            pltpu.VMEM((Mt,), jnp.int32),
            pltpu.VMEM((Mt,), jnp.float32),
            pltpu.VMEM((Mc,), jnp.int32),
            pltpu.VMEM((Mc,), jnp.int32),
            pltpu.VMEM((Mt,), jnp.float32),
            pltpu.VMEM((Mc,), jnp.float32),
            pltpu.VMEM((16 * Mc,), jnp.float32),
            pltpu.VMEM((Mc,), jnp.float32),
            pltpu.VMEM((Mc,), jnp.float32),
            pltpu.VMEM((CH,), jnp.int32),
            pltpu.VMEM((CH,), jnp.int32),
            pltpu.VMEM((CH,), jnp.float32),
            pltpu.VMEM_SHARED((8, Mc), jnp.float32),
            pltpu.SemaphoreType.DMA,
        ],
    )()
    return f(tp, head_idx, tail_idx, triple_labels, concept_labels, distances,
             vm_p, mm_p)


def _k5_body(cpvp_ref, lm_ref, gate_ref, probs_ref, cpv_ref, isc_ref):
    cpx = cpvp_ref[0, :, :V]
    lm = lm_ref[0]
    gcol = gate_ref[0, 0].reshape(L, 1)
    p = gcol * cpx + (1.0 - gcol) * lm
    probs_ref[0] = p
    cpv_ref[0] = cpx
    iota = lax.broadcasted_iota(jnp.int32, (L, V), 1)
    mP = jnp.max(p, axis=1, keepdims=True)
    mL = jnp.max(lm, axis=1, keepdims=True)
    iP = jnp.min(jnp.where(p == mP, iota, BIGI), axis=1)
    iL = jnp.min(jnp.where(lm == mL, iota, BIGI), axis=1)
    isc_ref[0, 0] = jnp.where(iP != iL, 1, 0).astype(jnp.int32)


def _k5(cpv_p, lm_probs, gate_bl):
    return pl.pallas_call(
        _k5_body,
        grid=(B,),
        out_shape=[jax.ShapeDtypeStruct((B, L, V), jnp.float32),
                   jax.ShapeDtypeStruct((B, L, V), jnp.float32),
                   jax.ShapeDtypeStruct((B, 1, L), jnp.int32)],
        in_specs=[pl.BlockSpec((1, L, VP), lambda b: (b, 0, 0)),
                  pl.BlockSpec((1, L, V), lambda b: (b, 0, 0)),
                  pl.BlockSpec((1, 1, L), lambda b: (b, 0, 0))],
        out_specs=[pl.BlockSpec((1, L, V), lambda b: (b, 0, 0)),
                   pl.BlockSpec((1, L, V), lambda b: (b, 0, 0)),
                   pl.BlockSpec((1, 1, L), lambda b: (b, 0, 0))],
    )(cpv_p, lm_probs, gate_bl)


def kernel(lm_hidden_states, lm_probs, triple_repr, triple_labels,
           concept_labels, distances, head_idx, tail_idx, vocab_map,
           map_mask, W_triple, W_gate, b_gate):
    h16 = lm_hidden_states.reshape(16, E)
    bg = b_gate.reshape(1, 1)
    h_proj, gate16, vm_p3, mm_p3 = _k1(
        h16, W_triple, W_gate.T, bg,
        vocab_map.reshape(B, 1, V), map_mask.reshape(B, 1, V))
    vm_p = vm_p3.reshape(B, VP)
    mm_p = mm_p3.reshape(B, VP)
    hp = h_proj.reshape(B, L, F)
    labels3 = triple_labels.reshape(B, 1, Mt)
    triple_prob = _k2(hp, triple_repr, labels3)
    cpv_p = _k34(triple_prob, head_idx, tail_idx, triple_labels,
                 concept_labels, distances, vm_p, mm_p)
    probs, cpv, isc = _k5(cpv_p, lm_probs, gate16.reshape(B, 1, L))
    gate = gate16.reshape(B, L, 1)
    return probs, gate, cpv, triple_prob, isc.reshape(B, L)


# speedup vs baseline: 1.1024x; 1.1024x over previous
import functools

import jax
import jax.numpy as jnp
from jax import lax
from jax.experimental import pallas as pl
from jax.experimental.pallas import tpu as pltpu
from jax.experimental.pallas import tpu_sc as plsc

B, L, E, Mt, Mc, V = 4, 4, 1024, 1024, 512, 50000
F = 3 * E
GAMMA = 0.8
VP = 50176
HALF = VP // 2
NV = HALF // 16
CH = HALF // 2
NCH = CH // 16
BM = 512
BIGI = 2 ** 30


def _k1_body(h_ref, wt_ref, wg_ref, bg_ref, hp_ref, gate_ref):
    h = h_ref[...]
    hp_ref[...] = jnp.dot(h, wt_ref[...], preferred_element_type=jnp.float32)
    z = jnp.dot(h, wg_ref[...], preferred_element_type=jnp.float32)
    gate_ref[...] = jax.nn.sigmoid(z + bg_ref[0, 0])


def _k1(h16, W_triple, Wg_t, bg):
    return pl.pallas_call(
        _k1_body,
        out_shape=[jax.ShapeDtypeStruct((16, F), jnp.float32),
                   jax.ShapeDtypeStruct((16, 1), jnp.float32)],
        in_specs=[pl.BlockSpec((16, E), lambda: (0, 0)),
                  pl.BlockSpec((E, F), lambda: (0, 0)),
                  pl.BlockSpec((E, 1), lambda: (0, 0)),
                  pl.BlockSpec(memory_space=pltpu.SMEM)],
        out_specs=[pl.BlockSpec((16, F), lambda: (0, 0)),
                   pl.BlockSpec((16, 1), lambda: (0, 0))],
    )(h16, W_triple, Wg_t, bg)


def _k2_body(hp_ref, tr_ref, lab_ref, out_ref):
    hp = hp_ref[0]
    tr = tr_ref[0]
    logits = lax.dot_general(hp, tr, (((1,), (1,)), ((), ())),
                             preferred_element_type=jnp.float32)
    prob = jax.nn.sigmoid(logits)
    lab = lab_ref[0]
    out_ref[0] = jnp.where(lab == -1, 0.0, prob)


def _k2(hp, triple_repr, labels3):
    return pl.pallas_call(
        _k2_body,
        grid=(B, Mt // BM),
        out_shape=jax.ShapeDtypeStruct((B, L, Mt), jnp.float32),
        in_specs=[pl.BlockSpec((1, L, F), lambda b, j: (b, 0, 0)),
                  pl.BlockSpec((1, BM, F), lambda b, j: (b, j, 0)),
                  pl.BlockSpec((1, 1, BM), lambda b, j: (b, 0, j))],
        out_specs=pl.BlockSpec((1, L, BM), lambda b, j: (b, 0, j)),
    )(hp, triple_repr, labels3)


def _k34_body(tp_hbm, head_hbm, tail_hbm, tlab_hbm, clab_hbm, dist_hbm,
              vm_hbm, mm_hbm,
              cpv_hbm,
              head_v, tail_v, tlab_v, tp_v, clab_v, dist_v, upd_v, ns_v,
              priv_v, acc_v,
              table_v, vm_v, mm_v, cv_v, shared_cp, sem):
    c = lax.axis_index("c")
    s = lax.axis_index("s")
    pair2 = c * 8 + s // 2
    half = s % 2
    b2 = pair2 // 4
    l2 = pair2 % 4
    base = pl.multiple_of(half * HALF, 32)

    @pl.when(s < 8)
    def _():
        pair = c * 8 + s
        b = pair // 4
        l = pair % 4
        d1 = pltpu.async_copy(head_hbm.at[b], head_v, sem)
        d2 = pltpu.async_copy(tail_hbm.at[b], tail_v, sem)
        d3 = pltpu.async_copy(tlab_hbm.at[b], tlab_v, sem)
        d4 = pltpu.async_copy(tp_hbm.at[b, l], tp_v, sem)
        d5 = pltpu.async_copy(clab_hbm.at[b], clab_v, sem)
        d6 = pltpu.async_copy(dist_hbm.at[b], dist_v, sem)
        d1.wait()
        d2.wait()
        d3.wait()
        d4.wait()
        d5.wait()
        d6.wait()
        one = jnp.ones((16,), jnp.float32)
        zero = jnp.zeros((16,), jnp.float32)
        lane = lax.iota(jnp.int32, 16)
        for i in range(Mc // 16):
            sl = pl.ds(i * 16, 16)
            f0 = jnp.where(dist_v[sl] == 0, one, zero)
            acc_v[sl] = f0
            ns_v[sl] = jnp.where(clab_v[sl] == -1, zero, f0)
        for _hop in range(2):
            for i in range(Mt // 16):
                sl = pl.ds(i * 16, 16)
                hs = plsc.load_gather(ns_v, [head_v[sl]])
                upd = hs * GAMMA + tp_v[sl]
                upd_v[sl] = jnp.where(tlab_v[sl] == -1, zero, upd)

            def zbody(i, carry):
                priv_v[pl.ds(i * 16, 16)] = zero
                return carry

            lax.fori_loop(0, 16 * Mc // 16, zbody, 0)

            def scbody(i, carry):
                sl = pl.ds(i * 16, 16)
                addr = lane * Mc + tail_v[sl]
                cur = plsc.load_gather(priv_v, [addr])
                plsc.store_scatter(priv_v, [addr],
                                   jnp.maximum(cur, upd_v[sl]))
                return carry

            lax.fori_loop(0, Mt // 16, scbody, 0)

            def cbody(i, carry):
                sl16 = pl.ds(i * 16, 16)
                m = priv_v[sl16]
                for j in range(1, 16):
                    m = jnp.maximum(m, priv_v[pl.ds(j * Mc + i * 16, 16)])
                o = jnp.where(clab_v[sl16] == -1, zero, m)
                ns_v[sl16] = o
                acc_v[sl16] = acc_v[sl16] + o
                return carry

            lax.fori_loop(0, Mc // 16, cbody, 0)
        mxv = acc_v[pl.ds(0, 16)]
        for i in range(1, Mc // 16):
            mxv = jnp.maximum(mxv, acc_v[pl.ds(i * 16, 16)])
        m = jnp.max(mxv)
        sv = jnp.zeros((16,), jnp.float32)
        for i in range(Mc // 16):
            sl = pl.ds(i * 16, 16)
            e = jnp.exp(acc_v[sl] - m)
            acc_v[sl] = e
            sv = sv + e
        inv_v = one / (zero + jnp.sum(sv))
        for i in range(Mc // 16):
            sl = pl.ds(i * 16, 16)
            acc_v[sl] = acc_v[sl] * inv_v
        pltpu.sync_copy(acc_v, shared_cp.at[s])

    plsc.subcore_barrier()

    pltpu.sync_copy(shared_cp.at[s // 2], table_v)
    zero = jnp.zeros((16,), jnp.float32)

    def inner(k, carry):
        sl = pl.ds(k * 16, 16)
        cpx = plsc.load_gather(table_v, [vm_v[sl]])
        cv_v[sl] = jnp.where(mm_v[sl] == 0, zero, cpx)
        return carry

    d_vm = pltpu.async_copy(vm_hbm.at[b2, pl.ds(base, CH)], vm_v, sem)
    d_mm = pltpu.async_copy(mm_hbm.at[b2, pl.ds(base, CH)], mm_v, sem)
    d_vm.wait()
    d_mm.wait()
    lax.fori_loop(0, NCH, inner, 0)
    pltpu.sync_copy(cv_v, cpv_hbm.at[b2, l2, pl.ds(base, CH)])
    base1 = pl.multiple_of(base + CH, 32)
    d_vm1 = pltpu.async_copy(vm_hbm.at[b2, pl.ds(base1, CH)], vm_v, sem)
    d_mm1 = pltpu.async_copy(mm_hbm.at[b2, pl.ds(base1, CH)], mm_v, sem)
    d_vm1.wait()
    d_mm1.wait()
    lax.fori_loop(0, NCH, inner, 0)
    pltpu.sync_copy(cv_v, cpv_hbm.at[b2, l2, pl.ds(base1, CH)])


def _k34(tp, head_idx, tail_idx, triple_labels, concept_labels, distances,
         vm_p, mm_p):
    mesh = plsc.VectorSubcoreMesh(core_axis_name="c", subcore_axis_name="s")
    f = functools.partial(
        pl.kernel, _k34_body, mesh=mesh,
        compiler_params=pltpu.CompilerParams(needs_layout_passes=False),
        out_type=jax.ShapeDtypeStruct((B, L, VP), jnp.float32),
        scratch_types=[
            pltpu.VMEM((Mt,), jnp.int32),
            pltpu.VMEM((Mt,), jnp.int32),
            pltpu.VMEM((Mt,), jnp.int32),
            pltpu.VMEM((Mt,), jnp.float32),
            pltpu.VMEM((Mc,), jnp.int32),
            pltpu.VMEM((Mc,), jnp.int32),
            pltpu.VMEM((Mt,), jnp.float32),
            pltpu.VMEM((Mc,), jnp.float32),
            pltpu.VMEM((16 * Mc,), jnp.float32),
            pltpu.VMEM((Mc,), jnp.float32),
            pltpu.VMEM((Mc,), jnp.float32),
            pltpu.VMEM((CH,), jnp.int32),
            pltpu.VMEM((CH,), jnp.int32),
            pltpu.VMEM((CH,), jnp.float32),
            pltpu.VMEM_SHARED((8, Mc), jnp.float32),
            pltpu.SemaphoreType.DMA,
        ],
    )()
    return f(tp, head_idx, tail_idx, triple_labels, concept_labels, distances,
             vm_p, mm_p)


def _k5_body(cpvp_ref, lm_ref, gate_ref, probs_ref, cpv_ref, isc_ref):
    cpx = cpvp_ref[0, :, :V]
    lm = lm_ref[0]
    gcol = gate_ref[0, 0].reshape(L, 1)
    p = gcol * cpx + (1.0 - gcol) * lm
    probs_ref[0] = p
    cpv_ref[0] = cpx
    iota = lax.broadcasted_iota(jnp.int32, (L, V), 1)
    mP = jnp.max(p, axis=1, keepdims=True)
    mL = jnp.max(lm, axis=1, keepdims=True)
    iP = jnp.min(jnp.where(p == mP, iota, BIGI), axis=1)
    iL = jnp.min(jnp.where(lm == mL, iota, BIGI), axis=1)
    isc_ref[0, 0] = jnp.where(iP != iL, 1, 0).astype(jnp.int32)


def _k5(cpv_p, lm_probs, gate_bl):
    return pl.pallas_call(
        _k5_body,
        grid=(B,),
        out_shape=[jax.ShapeDtypeStruct((B, L, V), jnp.float32),
                   jax.ShapeDtypeStruct((B, L, V), jnp.float32),
                   jax.ShapeDtypeStruct((B, 1, L), jnp.int32)],
        in_specs=[pl.BlockSpec((1, L, VP), lambda b: (b, 0, 0)),
                  pl.BlockSpec((1, L, V), lambda b: (b, 0, 0)),
                  pl.BlockSpec((1, 1, L), lambda b: (b, 0, 0))],
        out_specs=[pl.BlockSpec((1, L, V), lambda b: (b, 0, 0)),
                   pl.BlockSpec((1, L, V), lambda b: (b, 0, 0)),
                   pl.BlockSpec((1, 1, L), lambda b: (b, 0, 0))],
    )(cpv_p, lm_probs, gate_bl)


def kernel(lm_hidden_states, lm_probs, triple_repr, triple_labels,
           concept_labels, distances, head_idx, tail_idx, vocab_map,
           map_mask, W_triple, W_gate, b_gate):
    h16 = lm_hidden_states.reshape(16, E)
    bg = b_gate.reshape(1, 1)
    h_proj, gate16 = _k1(h16, W_triple, W_gate.T, bg)
    hp = h_proj.reshape(B, L, F)
    labels3 = triple_labels.reshape(B, 1, Mt)
    triple_prob = _k2(hp, triple_repr, labels3)
    vm_p = jnp.pad(vocab_map, ((0, 0), (0, VP - V)))
    mm_p = jnp.pad(map_mask, ((0, 0), (0, VP - V)))
    cpv_p = _k34(triple_prob, head_idx, tail_idx, triple_labels,
                 concept_labels, distances, vm_p, mm_p)
    probs, cpv, isc = _k5(cpv_p, lm_probs, gate16.reshape(B, 1, L))
    gate = gate16.reshape(B, L, 1)
    return probs, gate, cpv, triple_prob, isc.reshape(B, L)
